# baseline (speedup 1.0000x reference)
import jax
import jax.numpy as jnp
from jax.experimental import pallas as pl


def _elu(x):
    return jnp.where(x > 0, x, jnp.exp(x) - 1.0)


def _fused_gcn_kernel(x_ref, w1_ref, w2_ref, o_ref):
    x = x_ref[0]
    xb = x.astype(jnp.bfloat16)
    a = jnp.dot(xb, xb.T, preferred_element_type=jnp.float32)
    a = jnp.tanh(jax.nn.relu(a)).astype(jnp.bfloat16)
    h = x
    for w_ref in (w1_ref, w2_ref):
        agg = jnp.dot(a, h.astype(jnp.bfloat16),
                      preferred_element_type=jnp.float32)
        h = _elu(jnp.dot(agg, w_ref[...], preferred_element_type=jnp.float32))
    o_ref[0] = h


def kernel(X, W1, W2):
    Bv, NTv, Dv = X.shape
    out = pl.pallas_call(
        _fused_gcn_kernel,
        grid=(Bv,),
        in_specs=[
            pl.BlockSpec((1, NTv, Dv), lambda b: (b, 0, 0)),
            pl.BlockSpec((Dv, Dv), lambda b: (0, 0)),
            pl.BlockSpec((Dv, Dv), lambda b: (0, 0)),
        ],
        out_specs=pl.BlockSpec((1, NTv, Dv), lambda b: (b, 0, 0)),
        out_shape=jax.ShapeDtypeStruct((Bv, NTv, Dv), jnp.float32),
    )(X, W1, W2)
    return out


# speedup vs baseline: 1.1908x; 1.1908x over previous
import jax
import jax.numpy as jnp
from jax.experimental import pallas as pl


def _elu(x):
    return jnp.where(x > 0, x, jnp.exp(x) - 1.0)


def _fused_gcn_kernel(x_ref, w1_ref, w2_ref, o_ref):
    nb = x_ref.shape[0]
    xs = [x_ref[b] for b in range(nb)]
    as_ = []
    for x in xs:
        xb = x.astype(jnp.bfloat16)
        g = jnp.dot(xb, xb.T, preferred_element_type=jnp.float32)
        as_.append(jnp.tanh(jax.nn.relu(g)).astype(jnp.bfloat16))
    hs = list(xs)
    for w_ref in (w1_ref, w2_ref):
        w = w_ref[...]
        aggs = [jnp.dot(a, h.astype(jnp.bfloat16),
                        preferred_element_type=jnp.float32)
                for a, h in zip(as_, hs)]
        hs = [_elu(jnp.dot(agg, w, preferred_element_type=jnp.float32))
              for agg in aggs]
    for b in range(nb):
        o_ref[b] = hs[b]


def kernel(X, W1, W2):
    Bv, NTv, Dv = X.shape
    out = pl.pallas_call(
        _fused_gcn_kernel,
        out_shape=jax.ShapeDtypeStruct((Bv, NTv, Dv), jnp.float32),
    )(X, W1, W2)
    return out


# speedup vs baseline: 1.1964x; 1.0047x over previous
import jax
import jax.numpy as jnp
from jax.experimental import pallas as pl


def _elu(x):
    return jnp.where(x > 0, x, jnp.exp(x) - 1.0)


def _fused_gcn_kernel(x_ref, w1_ref, w2_ref, o_ref):
    nb = x_ref.shape[0]
    xs = [x_ref[b] for b in range(nb)]
    w1 = w1_ref[...]
    w2 = w2_ref[...]
    hws = [jnp.dot(x, w1, preferred_element_type=jnp.float32).astype(jnp.bfloat16)
           for x in xs]
    as_ = []
    for x in xs:
        xb = x.astype(jnp.bfloat16)
        g = jnp.dot(xb, xb.T, preferred_element_type=jnp.float32)
        as_.append(jnp.tanh(jax.nn.relu(g)).astype(jnp.bfloat16))
    h1s = [_elu(jnp.dot(a, hw, preferred_element_type=jnp.float32))
           for a, hw in zip(as_, hws)]
    hw2s = [jnp.dot(h1, w2, preferred_element_type=jnp.float32).astype(jnp.bfloat16)
            for h1 in h1s]
    h2s = [_elu(jnp.dot(a, hw, preferred_element_type=jnp.float32))
           for a, hw in zip(as_, hw2s)]
    for b in range(nb):
        o_ref[b] = h2s[b]


def kernel(X, W1, W2):
    Bv, NTv, Dv = X.shape
    out = pl.pallas_call(
        _fused_gcn_kernel,
        out_shape=jax.ShapeDtypeStruct((Bv, NTv, Dv), jnp.float32),
    )(X, W1, W2)
    return out


# speedup vs baseline: 1.1975x; 1.0009x over previous
import jax
import jax.numpy as jnp
from jax.experimental import pallas as pl
from jax.experimental.pallas import tpu as pltpu


def _elu(x):
    return jnp.where(x > 0, x, jnp.exp(x) - 1.0)


def _fused_gcn_kernel(x_ref, w1_ref, w2_ref, o_hbm, ov_ref, sem):
    nb = x_ref.shape[0]
    xs = [x_ref[b] for b in range(nb)]
    w1 = w1_ref[...]
    w2 = w2_ref[...]
    hws = [jnp.dot(x, w1, preferred_element_type=jnp.float32).astype(jnp.bfloat16)
           for x in xs]
    as_ = []
    for x in xs:
        xb = x.astype(jnp.bfloat16)
        g = jnp.dot(xb, xb.T, preferred_element_type=jnp.float32)
        as_.append(jnp.tanh(jax.nn.relu(g)).astype(jnp.bfloat16))
    h1s = [_elu(jnp.dot(a, hw, preferred_element_type=jnp.float32))
           for a, hw in zip(as_, hws)]
    hw2s = [jnp.dot(h1, w2, preferred_element_type=jnp.float32).astype(jnp.bfloat16)
            for h1 in h1s]
    copies = []
    for b in range(nb):
        h2 = _elu(jnp.dot(as_[b], hw2s[b], preferred_element_type=jnp.float32))
        ov_ref[b] = h2
        cp = pltpu.make_async_copy(ov_ref.at[b], o_hbm.at[b], sem.at[b])
        cp.start()
        copies.append(cp)
    for cp in copies:
        cp.wait()


def kernel(X, W1, W2):
    Bv, NTv, Dv = X.shape
    out = pl.pallas_call(
        _fused_gcn_kernel,
        out_shape=jax.ShapeDtypeStruct((Bv, NTv, Dv), jnp.float32),
        out_specs=pl.BlockSpec(memory_space=pl.ANY),
        scratch_shapes=[
            pltpu.VMEM((Bv, NTv, Dv), jnp.float32),
            pltpu.SemaphoreType.DMA((Bv,)),
        ],
    )(X, W1, W2)
    return out
